# baseline (speedup 1.0000x reference)
import jax
import jax.numpy as jnp
from jax import lax
from jax.experimental import pallas as pl
from jax.experimental.pallas import tpu as pltpu
from jax.experimental.pallas import tpu_sc as plsc

_N = 10000
_G = 64
_NT = 16
_W = 640
_SH = 624


def _seg_update(bbuf, vals, acc_e, acc_i, lane, off):
    b = bbuf[pl.ds(off, 16)]
    bn = bbuf[pl.ds(off + 1, 16)]
    m_end = (lane == 15) | (b != bn)
    m_int = (lane < 15) & (b != bn)
    plsc.addupdate_scatter(acc_e, [b], vals, mask=m_end)
    plsc.addupdate_scatter(acc_i, [bn], vals, mask=m_int)


def _body(x_hbm, b_hbm, ix_hbm, out_hbm, colbuf, bbuf, idxb, sem, sem2,
          sem3, sums, cnts, sums_i, cnts_i, sh_s, sh_c, t_s, t_c, obuf):
    wid = lax.axis_index("s")
    lane = lax.iota(jnp.int32, 16)
    zeros16 = jnp.zeros((16,), jnp.float32)
    for j in range(_G // 16):
        sums[pl.ds(j * 16, 16)] = zeros16
        cnts[pl.ds(j * 16, 16)] = zeros16
        sums_i[pl.ds(j * 16, 16)] = zeros16
        cnts_i[pl.ds(j * 16, 16)] = zeros16

    base = wid * _SH
    d_ix = pltpu.async_copy(ix_hbm.at[pl.ds(base, _W)], idxb, sem2)
    d_b = pltpu.async_copy(b_hbm.at[pl.ds(base, _W)], bbuf.at[pl.ds(0, _W)],
                           sem3)
    d_ix.wait()
    gather = pltpu.async_copy(x_hbm.at[idxb], colbuf, sem)
    d_b.wait()

    cc = (lane + 1).astype(jnp.float32)
    last = wid == _NT - 1

    def count_body(it, carry):
        for k in range(3):
            _seg_update(bbuf, cc, cnts, cnts_i, lane, it * 48 + k * 16)
        return carry

    lax.fori_loop(0, _SH // 48, count_body, 0)

    @pl.when(last)
    def _():
        _seg_update(bbuf, cc, cnts, cnts_i, lane, _SH)

    gather.wait()

    def sum_body(it, carry):
        for k in range(3):
            off = it * 48 + k * 16
            c = jnp.cumsum(colbuf[pl.ds(off, 16)])
            _seg_update(bbuf, c, sums, sums_i, lane, off)
        return carry

    lax.fori_loop(0, _SH // 48, sum_body, 0)

    @pl.when(last)
    def _():
        _seg_update(bbuf, jnp.cumsum(colbuf[pl.ds(_SH, 16)]), sums, sums_i,
                    lane, _SH)

    for j in range(_G // 16):
        s = pl.ds(j * 16, 16)
        sums[s] = sums[s] - sums_i[s]
        cnts[s] = cnts[s] - cnts_i[s]
    pltpu.sync_copy(sums, sh_s.at[pl.ds(wid * _G, _G)])
    pltpu.sync_copy(cnts, sh_c.at[pl.ds(wid * _G, _G)])
    plsc.subcore_barrier()

    @pl.when(wid < _G // 16)
    def _():
        pltpu.sync_copy(sh_s, t_s)
        pltpu.sync_copy(sh_c, t_c)
        col = wid * 16

        def red_body(r, accs):
            return (accs[0] + t_s[pl.ds(r * _G + col, 16)],
                    accs[1] + t_c[pl.ds(r * _G + col, 16)])

        acc_s, acc_c = lax.fori_loop(0, _NT, red_body, (zeros16, zeros16))
        obuf[pl.ds(0, 16)] = acc_s / acc_c
        pltpu.sync_copy(obuf, out_hbm.at[pl.ds(col, 16)])


@jax.jit
def _seg_mean(x, batch):
    mesh = plsc.VectorSubcoreMesh(
        core_axis_name="c", subcore_axis_name="s", num_cores=1)
    col_idx = (jnp.arange(_N, dtype=jnp.int32) * 128)
    f = pl.kernel(
        _body,
        out_type=jax.ShapeDtypeStruct((_G,), jnp.float32),
        mesh=mesh,
        compiler_params=pltpu.CompilerParams(needs_layout_passes=False),
        scratch_types=[
            pltpu.VMEM((_W,), jnp.float32),
            pltpu.VMEM((_W + 16,), jnp.int32),
            pltpu.VMEM((_W,), jnp.int32),
            pltpu.SemaphoreType.DMA,
            pltpu.SemaphoreType.DMA,
            pltpu.SemaphoreType.DMA,
            pltpu.VMEM((_G,), jnp.float32),
            pltpu.VMEM((_G,), jnp.float32),
            pltpu.VMEM((_G,), jnp.float32),
            pltpu.VMEM((_G,), jnp.float32),
            pltpu.VMEM_SHARED((_NT * _G,), jnp.float32),
            pltpu.VMEM_SHARED((_NT * _G,), jnp.float32),
            pltpu.VMEM((_NT * _G,), jnp.float32),
            pltpu.VMEM((_NT * _G,), jnp.float32),
            pltpu.VMEM((16,), jnp.float32),
        ],
    )
    return f(x.reshape(-1), batch, col_idx)


def kernel(x, edge_index, edge_attr, batch):
    out = _seg_mean(x, batch.astype(jnp.int32))
    return out[:, None]


# speedup vs baseline: 1.0038x; 1.0038x over previous
import jax
import jax.numpy as jnp
from jax import lax
from jax.experimental import pallas as pl
from jax.experimental.pallas import tpu as pltpu
from jax.experimental.pallas import tpu_sc as plsc

_N = 10000
_G = 64
_NT = 16
_W = 640
_SH = 624


def _body(x_hbm, b_hbm, out_hbm, colbuf, bbuf, idxb, sem, sums, cnts,
          sh_s, sh_c, t_s, t_c, obuf):
    wid = lax.axis_index("s")
    lane = lax.iota(jnp.int32, 16)
    zeros16 = jnp.zeros((16,), jnp.float32)
    for j in range(_G // 16):
        sums[pl.ds(j * 16, 16)] = zeros16
        cnts[pl.ds(j * 16, 16)] = zeros16

    base = jnp.minimum(wid * _SH, _N - _W)
    nv = jnp.where(wid == _NT - 1, _W // 16, _SH // 16)

    pltpu.sync_copy(b_hbm.at[pl.ds(base, _W)], bbuf.at[pl.ds(0, _W)])

    def idx_body(it, carry):
        idxb[pl.ds(it * 16, 16)] = (base + it * 16 + lane) * 128
        return carry

    lax.fori_loop(0, _W // 16, idx_body, 0)
    gather = pltpu.async_copy(x_hbm.at[idxb], colbuf, sem)

    cc = (lane + 1).astype(jnp.float32)

    def count_body(it, carry):
        off = it * 16
        b = bbuf[pl.ds(off, 16)]
        bn = bbuf[pl.ds(off + 1, 16)]
        m_end = (lane == 15) | (b != bn)
        m_int = (lane < 15) & (b != bn)
        plsc.addupdate_scatter(cnts, [b], cc, mask=m_end)
        plsc.addupdate_scatter(cnts, [bn], -cc, mask=m_int)
        return carry

    lax.fori_loop(0, nv, count_body, 0)
    gather.wait()

    def sum_body(it, carry):
        off = it * 16
        b = bbuf[pl.ds(off, 16)]
        bn = bbuf[pl.ds(off + 1, 16)]
        c = jnp.cumsum(colbuf[pl.ds(off, 16)])
        m_end = (lane == 15) | (b != bn)
        m_int = (lane < 15) & (b != bn)
        plsc.addupdate_scatter(sums, [b], c, mask=m_end)
        plsc.addupdate_scatter(sums, [bn], -c, mask=m_int)
        return carry

    lax.fori_loop(0, nv, sum_body, 0)

    pltpu.sync_copy(sums, sh_s.at[pl.ds(wid * _G, _G)])
    pltpu.sync_copy(cnts, sh_c.at[pl.ds(wid * _G, _G)])
    plsc.subcore_barrier()

    @pl.when(wid == 0)
    def _():
        pltpu.sync_copy(sh_s, t_s)
        pltpu.sync_copy(sh_c, t_c)

        def red_body(r, accs):
            return tuple(
                accs[j] + t_s[pl.ds(r * _G + j * 16, 16)] if j < _G // 16
                else accs[j] + t_c[pl.ds(r * _G + (j - _G // 16) * 16, 16)]
                for j in range(2 * (_G // 16))
            )

        accs = lax.fori_loop(0, _NT, red_body, (zeros16,) * (2 * (_G // 16)))
        for j in range(_G // 16):
            obuf[pl.ds(j * 16, 16)] = accs[j] / accs[j + _G // 16]
        pltpu.sync_copy(obuf, out_hbm)


@jax.jit
def _seg_mean(x, batch):
    mesh = plsc.VectorSubcoreMesh(
        core_axis_name="c", subcore_axis_name="s", num_cores=1)
    f = pl.kernel(
        _body,
        out_type=jax.ShapeDtypeStruct((_G,), jnp.float32),
        mesh=mesh,
        compiler_params=pltpu.CompilerParams(needs_layout_passes=False),
        scratch_types=[
            pltpu.VMEM((_W,), jnp.float32),
            pltpu.VMEM((_W + 16,), jnp.int32),
            pltpu.VMEM((_W,), jnp.int32),
            pltpu.SemaphoreType.DMA,
            pltpu.VMEM((_G,), jnp.float32),
            pltpu.VMEM((_G,), jnp.float32),
            pltpu.VMEM_SHARED((_NT * _G,), jnp.float32),
            pltpu.VMEM_SHARED((_NT * _G,), jnp.float32),
            pltpu.VMEM((_NT * _G,), jnp.float32),
            pltpu.VMEM((_NT * _G,), jnp.float32),
            pltpu.VMEM((_G,), jnp.float32),
        ],
    )
    return f(x.reshape(-1), batch)


def kernel(x, edge_index, edge_attr, batch):
    out = _seg_mean(x, batch.astype(jnp.int32))
    return out[:, None]
